# baseline (speedup 1.0000x reference)
import functools

import jax
import jax.numpy as jnp
from jax import lax
from jax.experimental import pallas as pl
from jax.experimental.pallas import tpu as pltpu
from jax.experimental.pallas import tpu_sc as plsc

_NC, _NS, _LANES = 2, 16, 16
_NW = _NC * _NS

_F32 = jnp.float32


def _sds(shape, dtype=_F32):
    return jax.ShapeDtypeStruct(shape, dtype)


def _rbf_feat(d, vmin, vmax, bins):
    step = (vmax - vmin) / (bins - 1)
    centers = vmin + lax.broadcasted_iota(jnp.int32, (1, bins), 1).astype(_F32) * step
    gamma = 1.0 / (step * step)
    diff = d - centers
    return jnp.exp(-gamma * diff * diff)


def _stats_accum(s_ref, t):
    s = jnp.concatenate(
        [jnp.sum(t, axis=0, keepdims=True),
         jnp.sum(t * t, axis=0, keepdims=True)], axis=0)
    pid = pl.program_id(0)

    @pl.when(pid == 0)
    def _():
        s_ref[0] = s

    @pl.when(pid != 0)
    def _():
        s_ref[0] = s_ref[0] + s


def _bn_apply(t, stats, g, bt, n):
    st = jnp.sum(stats, axis=0)
    mean = st[0:1, :] / n
    var = st[1:2, :] / n - mean * mean
    rstd = lax.rsqrt(var + 1e-5)
    return (t - mean) * rstd * g + bt


def _mm_stats(x, w, b, feat, block):
    n = x.shape[0]
    kin = x.shape[1]
    kw = w.shape[0]
    dout = w.shape[1]

    def body(x_ref, w_ref, b_ref, o_ref, s_ref):
        xv = x_ref[...]
        if feat == "rbf_r":
            d = jnp.sqrt(jnp.sum(xv * xv, axis=1, keepdims=True))
            xv = _rbf_feat(d, 0.0, 6.0, 80)
        elif feat == "rbf_a":
            xv = _rbf_feat(xv, -1.0, 1.0, 40)
        t = jnp.dot(xv, w_ref[...], preferred_element_type=_F32) + b_ref[...]
        o_ref[...] = t
        _stats_accum(s_ref, t)

    return pl.pallas_call(
        body,
        grid=(n // block,),
        in_specs=[pl.BlockSpec((block, kin), lambda i: (i, 0)),
                  pl.BlockSpec((kw, dout), lambda i: (0, 0)),
                  pl.BlockSpec((1, dout), lambda i: (0, 0))],
        out_specs=[pl.BlockSpec((block, dout), lambda i: (i, 0)),
                   pl.BlockSpec((1, 2, dout), lambda i: (0, 0, 0))],
        out_shape=[_sds((n, dout)), _sds((1, 2, dout))],
    )(x, w, b)


def _bnact_mm_stats(t1, st1, g1, bt1, w, b, block):
    n, din = t1.shape
    dout = w.shape[1]
    p = st1.shape[0]

    def body(t_ref, st_ref, g_ref, bt_ref, w_ref, b_ref, o_ref, s_ref):
        a = jax.nn.silu(_bn_apply(t_ref[...], st_ref[...], g_ref[...],
                                  bt_ref[...], n))
        t2 = jnp.dot(a, w_ref[...], preferred_element_type=_F32) + b_ref[...]
        o_ref[...] = t2
        _stats_accum(s_ref, t2)

    return pl.pallas_call(
        body,
        grid=(n // block,),
        in_specs=[pl.BlockSpec((block, din), lambda i: (i, 0)),
                  pl.BlockSpec((p, 2, din), lambda i: (0, 0, 0)),
                  pl.BlockSpec((1, din), lambda i: (0, 0)),
                  pl.BlockSpec((1, din), lambda i: (0, 0)),
                  pl.BlockSpec((din, dout), lambda i: (0, 0)),
                  pl.BlockSpec((1, dout), lambda i: (0, 0))],
        out_specs=[pl.BlockSpec((block, dout), lambda i: (i, 0)),
                   pl.BlockSpec((1, 2, dout), lambda i: (0, 0, 0))],
        out_shape=[_sds((n, dout)), _sds((1, 2, dout))],
    )(t1, st1, g1, bt1, w, b)


def _bnact(t, st, g, bt, block, residual=None):
    n, d = t.shape
    p = st.shape[0]

    def body(*refs):
        if residual is not None:
            t_ref, st_ref, g_ref, bt_ref, x_ref, o_ref = refs
        else:
            t_ref, st_ref, g_ref, bt_ref, o_ref = refs
        a = jax.nn.silu(_bn_apply(t_ref[...], st_ref[...], g_ref[...],
                                  bt_ref[...], n))
        if residual is not None:
            a = x_ref[...] + a
        o_ref[...] = a

    in_specs = [pl.BlockSpec((block, d), lambda i: (i, 0)),
                pl.BlockSpec((p, 2, d), lambda i: (0, 0, 0)),
                pl.BlockSpec((1, d), lambda i: (0, 0)),
                pl.BlockSpec((1, d), lambda i: (0, 0))]
    args = [t, st, g, bt]
    if residual is not None:
        in_specs.append(pl.BlockSpec((block, d), lambda i: (i, 0)))
        args.append(residual)
    return pl.pallas_call(
        body,
        grid=(n // block,),
        in_specs=in_specs,
        out_specs=pl.BlockSpec((block, d), lambda i: (i, 0)),
        out_shape=_sds((n, d)),
    )(*args)


def _mm3(x, w1, b1, w2, b2, w3, b3, block):
    n, kin = x.shape
    d1, d2, d3 = w1.shape[1], w2.shape[1], w3.shape[1]

    def body(x_ref, w1_ref, b1_ref, w2_ref, b2_ref, w3_ref, b3_ref,
             o1_ref, o2_ref, o3_ref):
        xv = x_ref[...]
        o1_ref[...] = jnp.dot(xv, w1_ref[...], preferred_element_type=_F32) + b1_ref[...]
        o2_ref[...] = jnp.dot(xv, w2_ref[...], preferred_element_type=_F32) + b2_ref[...]
        o3_ref[...] = jnp.dot(xv, w3_ref[...], preferred_element_type=_F32) + b3_ref[...]

    return pl.pallas_call(
        body,
        grid=(n // block,),
        in_specs=[pl.BlockSpec((block, kin), lambda i: (i, 0)),
                  pl.BlockSpec((kin, d1), lambda i: (0, 0)),
                  pl.BlockSpec((1, d1), lambda i: (0, 0)),
                  pl.BlockSpec((kin, d2), lambda i: (0, 0)),
                  pl.BlockSpec((1, d2), lambda i: (0, 0)),
                  pl.BlockSpec((kin, d3), lambda i: (0, 0)),
                  pl.BlockSpec((1, d3), lambda i: (0, 0))],
        out_specs=[pl.BlockSpec((block, d1), lambda i: (i, 0)),
                   pl.BlockSpec((block, d2), lambda i: (i, 0)),
                   pl.BlockSpec((block, d3), lambda i: (i, 0))],
        out_shape=[_sds((n, d1)), _sds((n, d2)), _sds((n, d3))],
    )(x, w1, b1, w2, b2, w3, b3)


def _mm1(x, w, b, block):
    n, kin = x.shape
    dout = w.shape[1]

    def body(x_ref, w_ref, b_ref, o_ref):
        o_ref[...] = (jnp.dot(x_ref[...], w_ref[...],
                              preferred_element_type=_F32) + b_ref[...])

    return pl.pallas_call(
        body,
        grid=(n // block,),
        in_specs=[pl.BlockSpec((block, kin), lambda i: (i, 0)),
                  pl.BlockSpec((kin, dout), lambda i: (0, 0)),
                  pl.BlockSpec((1, dout), lambda i: (0, 0))],
        out_specs=pl.BlockSpec((block, dout), lambda i: (i, 0)),
        out_shape=_sds((n, dout)),
    )(x, w, b)


def _combine(xsu, acc, block):
    n = xsu.shape[0]

    def body(x_ref, a_ref, u_ref, s_ref):
        a = a_ref[...]
        tot = a[0] + a[1]
        ssum = tot[:, :64]
        csum = tot[:, 64:]
        u = x_ref[...] + csum / (ssum + 1e-6)
        u_ref[...] = u
        _stats_accum(s_ref, u)

    return pl.pallas_call(
        body,
        grid=(n // block,),
        in_specs=[pl.BlockSpec((block, 64), lambda i: (i, 0)),
                  pl.BlockSpec((2, block, 128), lambda i: (0, i, 0))],
        out_specs=[pl.BlockSpec((block, 64), lambda i: (i, 0)),
                   pl.BlockSpec((1, 2, 64), lambda i: (0, 0, 0))],
        out_shape=[_sds((n, 64)), _sds((1, 2, 64))],
    )(xsu, acc)


def _pool_fc(h, w, b, block):
    n = h.shape[0]
    grid = n // block

    def body(h_ref, w_ref, b_ref, o_ref, acc_ref):
        pid = pl.program_id(0)
        s = jnp.sum(h_ref[...], axis=0, keepdims=True)

        @pl.when(pid == 0)
        def _():
            acc_ref[...] = s

        @pl.when(pid != 0)
        def _():
            acc_ref[...] = acc_ref[...] + s

        @pl.when(pid == grid - 1)
        def _():
            pooled = acc_ref[...] / n
            o_ref[...] = (jnp.dot(pooled, w_ref[...],
                                  preferred_element_type=_F32) + b_ref[...])

    return pl.pallas_call(
        body,
        grid=(grid,),
        in_specs=[pl.BlockSpec((block, 64), lambda i: (i, 0)),
                  pl.BlockSpec((64, 1), lambda i: (0, 0)),
                  pl.BlockSpec((1, 1), lambda i: (0, 0))],
        out_specs=pl.BlockSpec((1, 1), lambda i: (0, 0)),
        out_shape=_sds((1, 1)),
        scratch_shapes=[pltpu.VMEM((1, 64), _F32)],
    )(h, w, b)


def _mesh():
    return plsc.VectorSubcoreMesh(core_axis_name="c", subcore_axis_name="s",
                                  num_cores=_NC, num_subcores=_NS)


def _zero_vmem2d(ref, rows, cols):
    zero16 = jnp.zeros((16,), _F32)
    nch = cols // 16

    def zb(i, _):
        ref[i // nch, pl.ds((i % nch) * 16, 16)] = zero16
        return 0

    lax.fori_loop(0, rows * nch, zb, 0)


def _edge_compute(gsd, gdst, gyw, mbuf, pay, k_rows, carry0):
    def row(rw, carry):
        new = list(carry)
        for j in range(4):
            sl = pl.ds(j * 16, 16)
            sl_hi = pl.ds(64 + j * 16, 16)
            mv = gsd[rw, sl] + gdst[rw, sl] + gyw[rw, sl]
            mbuf[rw, sl] = mv
            sg = 1.0 / (1.0 + jnp.exp(-mv))
            pay[rw, sl] = sg
            pay[rw, sl_hi] = gsd[rw, sl_hi] * sg
            new[j] = new[j] + mv
            new[4 + j] = new[4 + j] + mv * mv
        return tuple(new)

    return lax.fori_loop(0, k_rows, row, carry0)


def _fold_stats(stv, carry):
    for j in range(4):
        sl = pl.ds(j * 16, 16)
        stv[0, sl] = stv[0, sl] + carry[j]
        stv[1, sl] = stv[1, sl] + carry[4 + j]


_ZCARRY = None


def _zcarry():
    z = jnp.zeros((16,), _F32)
    return (z,) * 8


def _sc_crystal(tsd, tdst, yw, src, dst):
    n = tsd.shape[0]
    e = yw.shape[0]
    epw = e // _NW
    K = 40
    nit = epw // K
    rpt = 640
    nacc = _NS * rpt
    last = n - rpt * (_NS - 1)
    ZR = 40

    @functools.partial(
        pl.kernel,
        out_type=[_sds((e, 64)), _sds((_NC, n, 128)), _sds((_NW, 2, 64))],
        mesh=_mesh(),
        scratch_types=[
            pltpu.VMEM((K,), jnp.int32),
            pltpu.VMEM((K,), jnp.int32),
            pltpu.VMEM((K, 128), _F32),
            pltpu.VMEM((K, 128), _F32),
            pltpu.VMEM((K, 64), _F32),
            pltpu.VMEM((K, 64), _F32),
            pltpu.VMEM((K, 128), _F32),
            pltpu.VMEM((ZR, 128), _F32),
            pltpu.VMEM((2, 64), _F32),
            pltpu.VMEM_SHARED((nacc, 128), _F32),
            pltpu.SemaphoreType.DMA,
            pltpu.SemaphoreType.DMA,
        ],
    )
    def k(tsd_h, tdst_h, yw_h, src_h, dst_h, m_h, acc_h, st_h,
          isrc, idst, gsd, gdst, gyw, mbuf, pay, zbuf, stv, accs, sem1, sem2):
        cid = lax.axis_index("c")
        sid = lax.axis_index("s")
        wid = sid * _NC + cid
        r0 = sid * rpt

        _zero_vmem2d(zbuf, ZR, 128)
        _zero_vmem2d(stv, 2, 64)

        def zs(i, _):
            pltpu.sync_copy(zbuf, accs.at[pl.ds(r0 + i * ZR, ZR)])
            return 0

        lax.fori_loop(0, rpt // ZR, zs, 0)
        plsc.subcore_barrier()

        def it(i, _):
            base = wid * epw + i * K
            pltpu.sync_copy(src_h.at[pl.ds(base, K)], isrc)
            pltpu.sync_copy(dst_h.at[pl.ds(base, K)], idst)
            c1 = pltpu.async_copy(tsd_h.at[isrc], gsd, sem1)
            c2 = pltpu.async_copy(tdst_h.at[idst], gdst, sem2)
            pltpu.sync_copy(yw_h.at[pl.ds(base, K)], gyw)
            c1.wait()
            c2.wait()
            carry = _edge_compute(gsd, gdst, gyw, mbuf, pay, K, _zcarry())
            _fold_stats(stv, carry)
            pltpu.sync_copy(mbuf, m_h.at[pl.ds(base, K)])
            pltpu.sync_copy(pay, accs.at[idst], add=True)
            return 0

        lax.fori_loop(0, nit, it, 0)
        pltpu.sync_copy(stv, st_h.at[wid])
        plsc.subcore_barrier()

        @pl.when(sid < _NS - 1)
        def _():
            pltpu.sync_copy(accs.at[pl.ds(r0, rpt)],
                            acc_h.at[cid, pl.ds(r0, rpt)])

        @pl.when(sid == _NS - 1)
        def _():
            pltpu.sync_copy(accs.at[pl.ds(r0, last)],
                            acc_h.at[cid, pl.ds(r0, last)])

    return k(tsd, tdst, yw, src, dst)


def _sc_line_a(tsd, tdst, zw, lsrc, ldst):
    n = tsd.shape[0]
    e = zw.shape[0]
    epw = e // _NW
    K = 200
    nit = epw // K

    @functools.partial(
        pl.kernel,
        out_type=[_sds((e, 64)), _sds((e, 128)), _sds((_NW, 2, 64))],
        mesh=_mesh(),
        scratch_types=[
            pltpu.VMEM((K,), jnp.int32),
            pltpu.VMEM((K,), jnp.int32),
            pltpu.VMEM((K, 128), _F32),
            pltpu.VMEM((K, 128), _F32),
            pltpu.VMEM((K, 64), _F32),
            pltpu.VMEM((K, 64), _F32),
            pltpu.VMEM((K, 128), _F32),
            pltpu.VMEM((2, 64), _F32),
            pltpu.SemaphoreType.DMA,
            pltpu.SemaphoreType.DMA,
        ],
    )
    def k(tsd_h, tdst_h, zw_h, lsrc_h, ldst_h, m_h, pay_h, st_h,
          isrc, idst, gsd, gdst, gyw, mbuf, pay, stv, sem1, sem2):
        cid = lax.axis_index("c")
        sid = lax.axis_index("s")
        wid = sid * _NC + cid

        _zero_vmem2d(stv, 2, 64)

        def it(i, _):
            base = wid * epw + i * K
            pltpu.sync_copy(lsrc_h.at[pl.ds(base, K)], isrc)
            pltpu.sync_copy(ldst_h.at[pl.ds(base, K)], idst)
            c1 = pltpu.async_copy(tsd_h.at[isrc], gsd, sem1)
            c2 = pltpu.async_copy(tdst_h.at[idst], gdst, sem2)
            pltpu.sync_copy(zw_h.at[pl.ds(base, K)], gyw)
            c1.wait()
            c2.wait()
            carry = _edge_compute(gsd, gdst, gyw, mbuf, pay, K, _zcarry())
            _fold_stats(stv, carry)
            pltpu.sync_copy(mbuf, m_h.at[pl.ds(base, K)])
            pltpu.sync_copy(pay, pay_h.at[pl.ds(base, K)])
            return 0

        lax.fori_loop(0, nit, it, 0)
        pltpu.sync_copy(stv, st_h.at[wid])

    return k(tsd, tdst, zw, lsrc, ldst)


def _sc_line_b(pay, ldst, nseg):
    e = pay.shape[0]
    epw = e // _NW
    SPP = 6400
    npass = nseg // SPP
    ACCR = SPP + 128
    stripe = ACCR // _NS
    frt = SPP // _NS
    CK = 192
    NJ = CK // 16
    ZR = 24
    CAP = epw + 256

    @functools.partial(
        pl.kernel,
        out_type=[_sds((_NC, nseg, 128))],
        mesh=_mesh(),
        compiler_params=pltpu.CompilerParams(needs_layout_passes=False),
        scratch_types=[
            pltpu.VMEM((epw,), jnp.int32),
            pltpu.VMEM((CAP,), jnp.int32),
            pltpu.VMEM((CAP,), jnp.int32),
            pltpu.VMEM((CK,), jnp.int32),
            pltpu.VMEM((CK,), jnp.int32),
            pltpu.VMEM((CK, 128), _F32),
            pltpu.VMEM((ZR, 128), _F32),
            pltpu.VMEM_SHARED((ACCR, 128), _F32),
            pltpu.SemaphoreType.DMA,
        ],
    )
    def k(pay_h, ldst_h, acc_h,
          ldst_v, cid_b, cdst_b, idb, dstb, rows, zbuf, accs, sem):
        cid = lax.axis_index("c")
        sid = lax.axis_index("s")
        wid = sid * _NC + cid

        _zero_vmem2d(zbuf, ZR, 128)
        pltpu.sync_copy(ldst_h.at[pl.ds(wid * epw, epw)], ldst_v)

        z16i = jnp.zeros((16,), jnp.int32)
        tr16 = jnp.full((16,), SPP, jnp.int32)
        lanes = lax.iota(jnp.int32, 16)

        for p in range(npass):
            lo = p * SPP
            hi = lo + SPP

            def zs(i, _):
                pltpu.sync_copy(zbuf, accs.at[pl.ds(sid * stripe + i * ZR, ZR)])
                return 0

            lax.fori_loop(0, stripe // ZR, zs, 0)
            plsc.subcore_barrier()

            def comp(rw, cur_v):
                v = ldst_v[pl.ds(rw * 16, 16)]
                mk = (v >= lo) & (v < hi)
                mki = mk.astype(jnp.int32)
                cum = jnp.cumsum(mki)
                pos = jnp.where(mk, cur_v + cum - 1, CAP - 1)
                ids = (wid * epw + rw * 16) + lanes
                plsc.store_scatter(cdst_b, [pos], v - lo)
                plsc.store_scatter(cid_b, [pos], ids)
                return cur_v + plsc.all_reduce_population_count(mk)

            cur_v = lax.fori_loop(0, epw // 16, comp,
                                  jnp.zeros((16,), jnp.int32))
            cur = cur_v[0]

            for j in range(NJ):
                cid_b[pl.ds(cur + j * 16, 16)] = z16i
                cdst_b[pl.ds(cur + j * 16, 16)] = tr16

            nch = (cur + (CK - 1)) // CK

            def chunk(ci, _):
                for j in range(NJ):
                    sl = pl.ds(j * 16, 16)
                    idb[sl] = cid_b[pl.ds(ci * CK + j * 16, 16)]
                    dstb[sl] = cdst_b[pl.ds(ci * CK + j * 16, 16)]
                pltpu.async_copy(pay_h.at[idb], rows, sem).wait()
                pltpu.sync_copy(rows, accs.at[dstb], add=True)
                return 0

            lax.fori_loop(0, nch, chunk, 0)
            plsc.subcore_barrier()

            pltpu.sync_copy(accs.at[pl.ds(sid * frt, frt)],
                            acc_h.at[cid, pl.ds(lo + sid * frt, frt)])
            plsc.subcore_barrier()

    return k(pay, ldst)


def _row(v):
    return v.reshape(1, -1)


def _egc(xf, yf, s_idx, d_idx, pp, graph, bx, by):
    wsd = jnp.concatenate([pp["Wsg"], pp["Wdu"]], axis=1)
    bsd = _row(jnp.concatenate([pp["bsg"], pp["bdu"]]))
    zw64 = jnp.zeros((64, 64), _F32)
    zb64 = jnp.zeros((64,), _F32)
    wdg = jnp.concatenate([pp["Wdg"], zw64], axis=1)
    bdg = _row(jnp.concatenate([pp["bdg"], zb64]))
    tsd, tdst, xsu = _mm3(xf, wsd, bsd, wdg, bdg,
                          pp["Wsu"], _row(pp["bsu"]), bx)
    yw = _mm1(yf, pp["Weg"], _row(pp["beg"]), by)
    if graph == "crystal":
        m, acc, mstats = _sc_crystal(tsd, tdst, yw, s_idx, d_idx)
    else:
        m, pay, mstats = _sc_line_a(tsd, tdst, yw, s_idx, d_idx)
        acc = _sc_line_b(pay, d_idx, xf.shape[0])[0]
    u, ustats = _combine(xsu, acc, bx)
    x_new = _bnact(u, ustats, _row(pp["gn"]), _row(pp["bn_"]), bx, residual=xf)
    y_new = _bnact(m, mstats, _row(pp["ge"]), _row(pp["be_"]), by, residual=yf)
    return x_new, y_new


def kernel(x, r, angle_h, edge_index, line_edge_index, params):
    p = params
    src = edge_index[0].astype(jnp.int32)
    dst = edge_index[1].astype(jnp.int32)
    lsrc = line_edge_index[0].astype(jnp.int32)
    ldst = line_edge_index[1].astype(jnp.int32)

    BN, BE, BL = 2000, 2000, 2000

    t, st = _mm_stats(x, p["atom"]["W"], _row(p["atom"]["b"]), "none", BN)
    h = _bnact(t, st, _row(p["atom"]["g"]), _row(p["atom"]["bt"]), BN)

    t, st = _mm_stats(r, p["edge1"]["W"], _row(p["edge1"]["b"]), "rbf_r", BE)
    t, st = _bnact_mm_stats(t, st, _row(p["edge1"]["g"]), _row(p["edge1"]["bt"]),
                            p["edge2"]["W"], _row(p["edge2"]["b"]), BE)
    y = _bnact(t, st, _row(p["edge2"]["g"]), _row(p["edge2"]["bt"]), BE)

    t, st = _mm_stats(angle_h.reshape(-1, 1), p["ang1"]["W"],
                      _row(p["ang1"]["b"]), "rbf_a", BL)
    t, st = _bnact_mm_stats(t, st, _row(p["ang1"]["g"]), _row(p["ang1"]["bt"]),
                            p["ang2"]["W"], _row(p["ang2"]["b"]), BL)
    z = _bnact(t, st, _row(p["ang2"]["g"]), _row(p["ang2"]["bt"]), BL)

    for lp in p["alignn"]:
        h, y = _egc(h, y, src, dst, lp["node"], "crystal", BN, BE)
        y, z = _egc(y, z, lsrc, ldst, lp["edge"], "line", BE, BL)
    for lp in p["gcn"]:
        h, y = _egc(h, y, src, dst, lp, "crystal", BN, BE)

    out = _pool_fc(h, p["fc_W"], _row(p["fc_b"]), BN)
    return jnp.squeeze(out)


# speedup vs baseline: 1.9638x; 1.9638x over previous
import functools

import jax
import jax.numpy as jnp
from jax import lax
from jax.experimental import pallas as pl
from jax.experimental.pallas import tpu as pltpu
from jax.experimental.pallas import tpu_sc as plsc

_NC, _NS, _LANES = 2, 16, 16
_NW = _NC * _NS

_F32 = jnp.float32


def _sds(shape, dtype=_F32):
    return jax.ShapeDtypeStruct(shape, dtype)


def _rbf_feat(d, vmin, vmax, bins):
    step = (vmax - vmin) / (bins - 1)
    centers = vmin + lax.broadcasted_iota(jnp.int32, (1, bins), 1).astype(_F32) * step
    gamma = 1.0 / (step * step)
    diff = d - centers
    return jnp.exp(-gamma * diff * diff)


def _stats_accum(s_ref, t):
    s = jnp.concatenate(
        [jnp.sum(t, axis=0, keepdims=True),
         jnp.sum(t * t, axis=0, keepdims=True)], axis=0)
    pid = pl.program_id(0)

    @pl.when(pid == 0)
    def _():
        s_ref[0] = s

    @pl.when(pid != 0)
    def _():
        s_ref[0] = s_ref[0] + s


def _bn_apply(t, stats, g, bt, n):
    st = jnp.sum(stats, axis=0)
    mean = st[0:1, :] / n
    var = st[1:2, :] / n - mean * mean
    rstd = lax.rsqrt(var + 1e-5)
    return (t - mean) * rstd * g + bt


def _mm_stats(x, w, b, feat, block):
    n = x.shape[0]
    kin = x.shape[1]
    kw = w.shape[0]
    dout = w.shape[1]

    def body(x_ref, w_ref, b_ref, o_ref, s_ref):
        xv = x_ref[...]
        if feat == "rbf_r":
            d = jnp.sqrt(jnp.sum(xv * xv, axis=1, keepdims=True))
            xv = _rbf_feat(d, 0.0, 6.0, 80)
        elif feat == "rbf_a":
            xv = _rbf_feat(xv, -1.0, 1.0, 40)
        t = jnp.dot(xv, w_ref[...], preferred_element_type=_F32) + b_ref[...]
        o_ref[...] = t
        _stats_accum(s_ref, t)

    return pl.pallas_call(
        body,
        grid=(n // block,),
        in_specs=[pl.BlockSpec((block, kin), lambda i: (i, 0)),
                  pl.BlockSpec((kw, dout), lambda i: (0, 0)),
                  pl.BlockSpec((1, dout), lambda i: (0, 0))],
        out_specs=[pl.BlockSpec((block, dout), lambda i: (i, 0)),
                   pl.BlockSpec((1, 2, dout), lambda i: (0, 0, 0))],
        out_shape=[_sds((n, dout)), _sds((1, 2, dout))],
    )(x, w, b)


def _bnact_mm_stats(t1, st1, g1, bt1, w, b, block):
    n, din = t1.shape
    dout = w.shape[1]
    p = st1.shape[0]

    def body(t_ref, st_ref, g_ref, bt_ref, w_ref, b_ref, o_ref, s_ref):
        a = jax.nn.silu(_bn_apply(t_ref[...], st_ref[...], g_ref[...],
                                  bt_ref[...], n))
        t2 = jnp.dot(a, w_ref[...], preferred_element_type=_F32) + b_ref[...]
        o_ref[...] = t2
        _stats_accum(s_ref, t2)

    return pl.pallas_call(
        body,
        grid=(n // block,),
        in_specs=[pl.BlockSpec((block, din), lambda i: (i, 0)),
                  pl.BlockSpec((p, 2, din), lambda i: (0, 0, 0)),
                  pl.BlockSpec((1, din), lambda i: (0, 0)),
                  pl.BlockSpec((1, din), lambda i: (0, 0)),
                  pl.BlockSpec((din, dout), lambda i: (0, 0)),
                  pl.BlockSpec((1, dout), lambda i: (0, 0))],
        out_specs=[pl.BlockSpec((block, dout), lambda i: (i, 0)),
                   pl.BlockSpec((1, 2, dout), lambda i: (0, 0, 0))],
        out_shape=[_sds((n, dout)), _sds((1, 2, dout))],
    )(t1, st1, g1, bt1, w, b)


def _bnact(t, st, g, bt, block, residual=None):
    n, d = t.shape
    p = st.shape[0]

    def body(*refs):
        if residual is not None:
            t_ref, st_ref, g_ref, bt_ref, x_ref, o_ref = refs
        else:
            t_ref, st_ref, g_ref, bt_ref, o_ref = refs
        a = jax.nn.silu(_bn_apply(t_ref[...], st_ref[...], g_ref[...],
                                  bt_ref[...], n))
        if residual is not None:
            a = x_ref[...] + a
        o_ref[...] = a

    in_specs = [pl.BlockSpec((block, d), lambda i: (i, 0)),
                pl.BlockSpec((p, 2, d), lambda i: (0, 0, 0)),
                pl.BlockSpec((1, d), lambda i: (0, 0)),
                pl.BlockSpec((1, d), lambda i: (0, 0))]
    args = [t, st, g, bt]
    if residual is not None:
        in_specs.append(pl.BlockSpec((block, d), lambda i: (i, 0)))
        args.append(residual)
    return pl.pallas_call(
        body,
        grid=(n // block,),
        in_specs=in_specs,
        out_specs=pl.BlockSpec((block, d), lambda i: (i, 0)),
        out_shape=_sds((n, d)),
    )(*args)


def _mm2(x, w1, b1, w2, b2, block):
    n, kin = x.shape
    d1, d2 = w1.shape[1], w2.shape[1]

    def body(x_ref, w1_ref, b1_ref, w2_ref, b2_ref, o1_ref, o2_ref):
        xv = x_ref[...]
        o1_ref[...] = jnp.dot(xv, w1_ref[...], preferred_element_type=_F32) + b1_ref[...]
        o2_ref[...] = jnp.dot(xv, w2_ref[...], preferred_element_type=_F32) + b2_ref[...]

    return pl.pallas_call(
        body,
        grid=(n // block,),
        in_specs=[pl.BlockSpec((block, kin), lambda i: (i, 0)),
                  pl.BlockSpec((kin, d1), lambda i: (0, 0)),
                  pl.BlockSpec((1, d1), lambda i: (0, 0)),
                  pl.BlockSpec((kin, d2), lambda i: (0, 0)),
                  pl.BlockSpec((1, d2), lambda i: (0, 0))],
        out_specs=[pl.BlockSpec((block, d1), lambda i: (i, 0)),
                   pl.BlockSpec((block, d2), lambda i: (i, 0))],
        out_shape=[_sds((n, d1)), _sds((n, d2))],
    )(x, w1, b1, w2, b2)


def _mm1(x, w, b, block):
    n, kin = x.shape
    dout = w.shape[1]

    def body(x_ref, w_ref, b_ref, o_ref):
        o_ref[...] = (jnp.dot(x_ref[...], w_ref[...],
                              preferred_element_type=_F32) + b_ref[...])

    return pl.pallas_call(
        body,
        grid=(n // block,),
        in_specs=[pl.BlockSpec((block, kin), lambda i: (i, 0)),
                  pl.BlockSpec((kin, dout), lambda i: (0, 0)),
                  pl.BlockSpec((1, dout), lambda i: (0, 0))],
        out_specs=pl.BlockSpec((block, dout), lambda i: (i, 0)),
        out_shape=_sds((n, dout)),
    )(x, w, b)


def _combine(tdst, acc, block):
    n = tdst.shape[0]

    def body(x_ref, a_ref, u_ref, s_ref):
        a = a_ref[...]
        tot = a[0] + a[1]
        ssum = tot[:, :64]
        csum = tot[:, 64:]
        u = x_ref[:, 64:] + csum / (ssum + 1e-6)
        u_ref[...] = u
        _stats_accum(s_ref, u)

    return pl.pallas_call(
        body,
        grid=(n // block,),
        in_specs=[pl.BlockSpec((block, 128), lambda i: (i, 0)),
                  pl.BlockSpec((2, block, 128), lambda i: (0, i, 0))],
        out_specs=[pl.BlockSpec((block, 64), lambda i: (i, 0)),
                   pl.BlockSpec((1, 2, 64), lambda i: (0, 0, 0))],
        out_shape=[_sds((n, 64)), _sds((1, 2, 64))],
    )(tdst, acc)


def _pool_fc(h, w, b, block):
    n = h.shape[0]
    grid = n // block

    def body(h_ref, w_ref, b_ref, o_ref, acc_ref):
        pid = pl.program_id(0)
        s = jnp.sum(h_ref[...], axis=0, keepdims=True)

        @pl.when(pid == 0)
        def _():
            acc_ref[...] = s

        @pl.when(pid != 0)
        def _():
            acc_ref[...] = acc_ref[...] + s

        @pl.when(pid == grid - 1)
        def _():
            pooled = acc_ref[...] / n
            o_ref[...] = (jnp.dot(pooled, w_ref[...],
                                  preferred_element_type=_F32) + b_ref[...])

    return pl.pallas_call(
        body,
        grid=(grid,),
        in_specs=[pl.BlockSpec((block, 64), lambda i: (i, 0)),
                  pl.BlockSpec((64, 1), lambda i: (0, 0)),
                  pl.BlockSpec((1, 1), lambda i: (0, 0))],
        out_specs=pl.BlockSpec((1, 1), lambda i: (0, 0)),
        out_shape=_sds((1, 1)),
        scratch_shapes=[pltpu.VMEM((1, 64), _F32)],
    )(h, w, b)


def _vgather(x, idx):
    return lax.gather(
        x, idx[:, None],
        lax.GatherDimensionNumbers(offset_dims=(), collapsed_slice_dims=(0,),
                                   start_index_map=(0,)),
        (1,), mode=lax.GatherScatterMode.PROMISE_IN_BOUNDS)


def _mesh():
    return plsc.VectorSubcoreMesh(core_axis_name="c", subcore_axis_name="s",
                                  num_cores=_NC, num_subcores=_NS)


def _zero_vmem2d(ref, rows, cols):
    zero16 = jnp.zeros((16,), _F32)
    nch = cols // 16

    def zb(i, _):
        ref[i // nch, pl.ds((i % nch) * 16, 16)] = zero16
        return 0

    lax.fori_loop(0, rows * nch, zb, 0)


def _edge_compute(gsd, gdst, gyw, pay, k_rows, carry0):
    def row(rw, carry):
        new = list(carry)
        for j in range(4):
            sl = pl.ds(j * 16, 16)
            sl_hi = pl.ds(64 + j * 16, 16)
            mv = gsd[rw, sl] + gdst[rw, sl] + gyw[rw, sl]
            gyw[rw, sl] = mv
            sg = 1.0 / (1.0 + jnp.exp(-mv))
            pay[rw, sl] = sg
            pay[rw, sl_hi] = gsd[rw, sl_hi] * sg
            new[j] = new[j] + mv
            new[4 + j] = new[4 + j] + mv * mv
        return tuple(new)

    return lax.fori_loop(0, k_rows, row, carry0)


def _fold_stats(stv, carry):
    for j in range(4):
        sl = pl.ds(j * 16, 16)
        stv[0, sl] = stv[0, sl] + carry[j]
        stv[1, sl] = stv[1, sl] + carry[4 + j]


_ZCARRY = None


def _zcarry():
    z = jnp.zeros((16,), _F32)
    return (z,) * 8


def _sc_crystal(tsd, tdst, yw, src, dst):
    n = tsd.shape[0]
    e = yw.shape[0]
    epw = e // _NW
    K = 40
    nit = epw // K
    rpt = 632
    nacc = _NS * rpt
    last = n - rpt * (_NS - 1)
    ZR = 8

    buf = lambda: [pltpu.VMEM((K,), jnp.int32),
                   pltpu.VMEM((K,), jnp.int32),
                   pltpu.VMEM((K, 128), _F32),
                   pltpu.VMEM((K, 128), _F32),
                   pltpu.VMEM((K, 64), _F32),
                   pltpu.VMEM((K, 128), _F32),
                   pltpu.SemaphoreType.DMA,
                   pltpu.SemaphoreType.DMA,
                   pltpu.SemaphoreType.DMA]

    @functools.partial(
        pl.kernel,
        out_type=[_sds((e, 64)), _sds((_NC, n, 128)), _sds((_NW, 2, 64))],
        mesh=_mesh(),
        scratch_types=buf() + buf() + [
            pltpu.VMEM((ZR, 128), _F32),
            pltpu.VMEM((2, 64), _F32),
            pltpu.VMEM_SHARED((nacc, 128), _F32),
        ],
    )
    def k(tsd_h, tdst_h, yw_h, src_h, dst_h, m_h, acc_h, st_h,
          isrc0, idst0, gsd0, gdst0, gyw0, pay0, s0a, s0b, s0c,
          isrc1, idst1, gsd1, gdst1, gyw1, pay1, s1a, s1b, s1c,
          zbuf, stv, accs):
        cid = lax.axis_index("c")
        sid = lax.axis_index("s")
        wid = sid * _NC + cid
        r0 = sid * rpt

        S = ((isrc0, idst0, gsd0, gdst0, gyw0, pay0, s0a, s0b, s0c),
             (isrc1, idst1, gsd1, gdst1, gyw1, pay1, s1a, s1b, s1c))

        _zero_vmem2d(zbuf, ZR, 128)
        _zero_vmem2d(stv, 2, 64)

        def zs(i, _):
            pltpu.sync_copy(zbuf, accs.at[pl.ds(r0 + i * ZR, ZR)])
            return 0

        lax.fori_loop(0, rpt // ZR, zs, 0)
        plsc.subcore_barrier()

        def fire(s, c):
            isrc, idst, gsd, gdst, gyw, _, sa, sb, sc = s
            base = wid * epw + c * K
            pltpu.sync_copy(src_h.at[pl.ds(base, K)], isrc)
            pltpu.sync_copy(dst_h.at[pl.ds(base, K)], idst)
            pltpu.async_copy(tsd_h.at[isrc], gsd, sa)
            pltpu.async_copy(tdst_h.at[idst], gdst, sb)
            pltpu.async_copy(yw_h.at[pl.ds(base, K)], gyw, sc)

        def drain(s, c):
            isrc, idst, gsd, gdst, gyw, pay, sa, sb, sc = s
            base = wid * epw + c * K
            pltpu.make_async_copy(tsd_h.at[isrc], gsd, sa).wait()
            pltpu.make_async_copy(tdst_h.at[idst], gdst, sb).wait()
            pltpu.make_async_copy(yw_h.at[pl.ds(base, K)], gyw, sc).wait()
            carry = _edge_compute(gsd, gdst, gyw, pay, K, _zcarry())
            _fold_stats(stv, carry)
            pltpu.sync_copy(gyw, m_h.at[pl.ds(base, K)])
            pltpu.sync_copy(pay, accs.at[idst], add=True)

        fire(S[0], 0)

        def it(kk, _):
            fire(S[1], 2 * kk + 1)
            drain(S[0], 2 * kk)
            fire(S[0], 2 * kk + 2)
            drain(S[1], 2 * kk + 1)
            return 0

        lax.fori_loop(0, (nit - 1) // 2, it, 0)
        drain(S[0], nit - 1)

        pltpu.sync_copy(stv, st_h.at[wid])
        plsc.subcore_barrier()

        @pl.when(sid < _NS - 1)
        def _():
            pltpu.sync_copy(accs.at[pl.ds(r0, rpt)],
                            acc_h.at[cid, pl.ds(r0, rpt)])

        @pl.when(sid == _NS - 1)
        def _():
            pltpu.sync_copy(accs.at[pl.ds(r0, last)],
                            acc_h.at[cid, pl.ds(r0, last)])

    return k(tsd, tdst, yw, src, dst)


def _sc_line_a(tsd, tdst, zw, lsrc, ldst):
    e = zw.shape[0]
    epw = e // _NW
    K = 80
    nit = epw // K

    buf = lambda: [pltpu.VMEM((K,), jnp.int32),
                   pltpu.VMEM((K,), jnp.int32),
                   pltpu.VMEM((K, 128), _F32),
                   pltpu.VMEM((K, 128), _F32),
                   pltpu.VMEM((K, 64), _F32),
                   pltpu.VMEM((K, 128), _F32),
                   pltpu.SemaphoreType.DMA,
                   pltpu.SemaphoreType.DMA,
                   pltpu.SemaphoreType.DMA]

    @functools.partial(
        pl.kernel,
        out_type=[_sds((e, 64)), _sds((e, 128)), _sds((_NW, 2, 64))],
        mesh=_mesh(),
        scratch_types=buf() + buf() + [pltpu.VMEM((2, 64), _F32)],
    )
    def k(tsd_h, tdst_h, zw_h, lsrc_h, ldst_h, m_h, pay_h, st_h,
          isrc0, idst0, gsd0, gdst0, gyw0, pay0, s0a, s0b, s0c,
          isrc1, idst1, gsd1, gdst1, gyw1, pay1, s1a, s1b, s1c,
          stv):
        cid = lax.axis_index("c")
        sid = lax.axis_index("s")
        wid = sid * _NC + cid

        S = ((isrc0, idst0, gsd0, gdst0, gyw0, pay0, s0a, s0b, s0c),
             (isrc1, idst1, gsd1, gdst1, gyw1, pay1, s1a, s1b, s1c))

        _zero_vmem2d(stv, 2, 64)

        def fire(s, c):
            isrc, idst, gsd, gdst, gyw, _, sa, sb, sc = s
            base = wid * epw + c * K
            pltpu.sync_copy(lsrc_h.at[pl.ds(base, K)], isrc)
            pltpu.sync_copy(ldst_h.at[pl.ds(base, K)], idst)
            pltpu.async_copy(tsd_h.at[isrc], gsd, sa)
            pltpu.async_copy(tdst_h.at[idst], gdst, sb)
            pltpu.async_copy(zw_h.at[pl.ds(base, K)], gyw, sc)

        def drain(s, c):
            isrc, idst, gsd, gdst, gyw, pay, sa, sb, sc = s
            base = wid * epw + c * K
            pltpu.make_async_copy(tsd_h.at[isrc], gsd, sa).wait()
            pltpu.make_async_copy(tdst_h.at[idst], gdst, sb).wait()
            pltpu.make_async_copy(zw_h.at[pl.ds(base, K)], gyw, sc).wait()
            carry = _edge_compute(gsd, gdst, gyw, pay, K, _zcarry())
            _fold_stats(stv, carry)
            pltpu.sync_copy(gyw, m_h.at[pl.ds(base, K)])
            pltpu.sync_copy(pay, pay_h.at[pl.ds(base, K)])

        fire(S[0], 0)

        def it(kk, _):
            fire(S[1], 2 * kk + 1)
            drain(S[0], 2 * kk)
            fire(S[0], 2 * kk + 2)
            drain(S[1], 2 * kk + 1)
            return 0

        lax.fori_loop(0, (nit - 1) // 2, it, 0)
        drain(S[0], nit - 1)

        pltpu.sync_copy(stv, st_h.at[wid])

    return k(tsd, tdst, zw, lsrc, ldst)


def _sc_line_b(pay, ldst, nseg):
    e = pay.shape[0]
    epw = e // _NW
    SPP = 6400
    npass = nseg // SPP
    ACCR = SPP + 128
    stripe = ACCR // _NS
    frt = SPP // _NS
    CK = 64
    NJ = CK // 16
    ZR = 51
    CAPS = epw + (npass + 2) * CK

    @functools.partial(
        pl.kernel,
        out_type=[_sds((_NC, nseg, 128))],
        mesh=_mesh(),
        compiler_params=pltpu.CompilerParams(needs_layout_passes=False),
        scratch_types=[
            pltpu.VMEM((epw,), jnp.int32),
            pltpu.VMEM((CAPS,), jnp.int32),
            pltpu.VMEM((CAPS,), jnp.int32),
            pltpu.VMEM((32,), jnp.int32),
            pltpu.VMEM((32,), jnp.int32),
            pltpu.VMEM((32,), jnp.int32),
            pltpu.VMEM((CK,), jnp.int32),
            pltpu.VMEM((CK,), jnp.int32),
            pltpu.VMEM((CK, 128), _F32),
            pltpu.VMEM((CK,), jnp.int32),
            pltpu.VMEM((CK,), jnp.int32),
            pltpu.VMEM((CK, 128), _F32),
            pltpu.VMEM((ZR, 128), _F32),
            pltpu.VMEM_SHARED((ACCR, 128), _F32),
            pltpu.SemaphoreType.DMA,
            pltpu.SemaphoreType.DMA,
        ],
    )
    def k(pay_h, ldst_h, acc_h,
          ldst_v, sid_b, sdst_b, cnt, cur, offs,
          idb0, dstb0, rows0, idb1, dstb1, rows1, zbuf, accs, semA, semB):
        cid = lax.axis_index("c")
        sid = lax.axis_index("s")
        wid = sid * _NC + cid

        S = ((idb0, dstb0, rows0, semA), (idb1, dstb1, rows1, semB))

        _zero_vmem2d(zbuf, ZR, 128)
        pltpu.sync_copy(ldst_h.at[pl.ds(wid * epw, epw)], ldst_v)

        z16i = jnp.zeros((16,), jnp.int32)
        big16 = jnp.full((16,), nseg, jnp.int32)
        lanes = lax.iota(jnp.int32, 16)

        cnt[pl.ds(0, 16)] = z16i
        cnt[pl.ds(16, 16)] = z16i
        cur[pl.ds(0, 16)] = z16i
        cur[pl.ds(16, 16)] = z16i

        def ranks(b):
            rank_lt = jnp.zeros((16,), jnp.int32)
            tot = jnp.ones((16,), jnp.int32)
            for kk in range(1, 16):
                down = _vgather(b, jnp.maximum(lanes - kk, 0))
                eq_d = ((down == b) & (lanes >= kk)).astype(jnp.int32)
                up = _vgather(b, jnp.minimum(lanes + kk, 15))
                eq_u = ((up == b) & (lanes <= 15 - kk)).astype(jnp.int32)
                rank_lt = rank_lt + eq_d
                tot = tot + eq_d + eq_u
            return rank_lt, tot

        def hist(rw, _):
            v = ldst_v[pl.ds(rw * 16, 16)]
            b = v // SPP
            _, tot = ranks(b)
            snap = plsc.load_gather(cnt, [b])
            plsc.store_scatter(cnt, [b], snap + tot)
            return 0

        lax.fori_loop(0, epw // 16, hist, 0)

        c0 = cnt[pl.ds(0, 16)]
        c1 = cnt[pl.ds(16, 16)]
        p0 = ((c0 + (CK - 1)) // CK) * CK
        p1 = ((c1 + (CK - 1)) // CK) * CK

        def prefix(x):
            for s in (1, 2, 4, 8):
                sh = _vgather(x, jnp.maximum(lanes - s, 0))
                x = x + jnp.where(lanes >= s, sh, 0)
            return x

        i0 = prefix(p0)
        i1 = prefix(p1) + i0[15]
        offs[pl.ds(0, 16)] = i0 - p0
        offs[pl.ds(16, 16)] = i1 - p1

        def pf(i, _):
            sid_b[pl.ds(i * 16, 16)] = z16i
            sdst_b[pl.ds(i * 16, 16)] = big16
            return 0

        lax.fori_loop(0, CAPS // 16, pf, 0)

        def place(rw, _):
            v = ldst_v[pl.ds(rw * 16, 16)]
            b = v // SPP
            rank_lt, tot = ranks(b)
            snap = plsc.load_gather(cur, [b])
            base = plsc.load_gather(offs, [b])
            pos = base + snap + rank_lt
            ids = (wid * epw + rw * 16) + lanes
            plsc.store_scatter(sid_b, [pos], ids)
            plsc.store_scatter(sdst_b, [pos], v)
            plsc.store_scatter(cur, [b], snap + tot)
            return 0

        lax.fori_loop(0, epw // 16, place, 0)

        for p in range(npass):
            lo = p * SPP

            def zs(i, _):
                pltpu.sync_copy(zbuf, accs.at[pl.ds(sid * stripe + i * ZR, ZR)])
                return 0

            lax.fori_loop(0, stripe // ZR, zs, 0)
            plsc.subcore_barrier()

            start = offs[pl.ds((p // 16) * 16, 16)][p % 16]
            nch = (cnt[pl.ds((p // 16) * 16, 16)][p % 16] + (CK - 1)) // CK
            half = (nch + 1) // 2

            def fire(s, ci):
                idb, dstb, rows, sem = s
                for j in range(NJ):
                    sl = pl.ds(j * 16, 16)
                    idb[sl] = sid_b[pl.ds(start + ci * CK + j * 16, 16)]
                    dv = sdst_b[pl.ds(start + ci * CK + j * 16, 16)]
                    dstb[sl] = jnp.minimum(dv - lo, SPP)
                pltpu.async_copy(pay_h.at[idb], rows, sem)

            def drain(s):
                idb, dstb, rows, sem = s
                pltpu.make_async_copy(pay_h.at[idb], rows, sem).wait()
                pltpu.sync_copy(rows, accs.at[dstb], add=True)

            @pl.when(nch > 0)
            def _():
                fire(S[0], 0)

            def it(kk, _):
                fire(S[1], 2 * kk + 1)
                drain(S[0])

                @pl.when(kk < half - 1)
                def _():
                    fire(S[0], 2 * kk + 2)

                drain(S[1])
                return 0

            lax.fori_loop(0, half, it, 0)
            plsc.subcore_barrier()

            pltpu.sync_copy(accs.at[pl.ds(sid * frt, frt)],
                            acc_h.at[cid, pl.ds(lo + sid * frt, frt)])
            plsc.subcore_barrier()

    return k(pay, ldst)


def _row(v):
    return v.reshape(1, -1)


def _egc(xf, yf, s_idx, d_idx, pp, graph, bx, by):
    wsd = jnp.concatenate([pp["Wsg"], pp["Wdu"]], axis=1)
    bsd = _row(jnp.concatenate([pp["bsg"], pp["bdu"]]))
    wdg = jnp.concatenate([pp["Wdg"], pp["Wsu"]], axis=1)
    bdg = _row(jnp.concatenate([pp["bdg"], pp["bsu"]]))
    tsd, tdst = _mm2(xf, wsd, bsd, wdg, bdg, bx)
    yw = _mm1(yf, pp["Weg"], _row(pp["beg"]), by)
    if graph == "crystal":
        m, acc, mstats = _sc_crystal(tsd, tdst, yw, s_idx, d_idx)
    else:
        m, pay, mstats = _sc_line_a(tsd, tdst, yw, s_idx, d_idx)
        acc = _sc_line_b(pay, d_idx, xf.shape[0])[0]
    u, ustats = _combine(tdst, acc, bx)
    x_new = _bnact(u, ustats, _row(pp["gn"]), _row(pp["bn_"]), bx, residual=xf)
    y_new = _bnact(m, mstats, _row(pp["ge"]), _row(pp["be_"]), by, residual=yf)
    return x_new, y_new


def kernel(x, r, angle_h, edge_index, line_edge_index, params):
    p = params
    src = edge_index[0].astype(jnp.int32)
    dst = edge_index[1].astype(jnp.int32)
    lsrc = line_edge_index[0].astype(jnp.int32)
    ldst = line_edge_index[1].astype(jnp.int32)

    BN, BE, BL = 2000, 2000, 2000

    t, st = _mm_stats(x, p["atom"]["W"], _row(p["atom"]["b"]), "none", BN)
    h = _bnact(t, st, _row(p["atom"]["g"]), _row(p["atom"]["bt"]), BN)

    t, st = _mm_stats(r, p["edge1"]["W"], _row(p["edge1"]["b"]), "rbf_r", BE)
    t, st = _bnact_mm_stats(t, st, _row(p["edge1"]["g"]), _row(p["edge1"]["bt"]),
                            p["edge2"]["W"], _row(p["edge2"]["b"]), BE)
    y = _bnact(t, st, _row(p["edge2"]["g"]), _row(p["edge2"]["bt"]), BE)

    t, st = _mm_stats(angle_h.reshape(-1, 1), p["ang1"]["W"],
                      _row(p["ang1"]["b"]), "rbf_a", BL)
    t, st = _bnact_mm_stats(t, st, _row(p["ang1"]["g"]), _row(p["ang1"]["bt"]),
                            p["ang2"]["W"], _row(p["ang2"]["b"]), BL)
    z = _bnact(t, st, _row(p["ang2"]["g"]), _row(p["ang2"]["bt"]), BL)

    for lp in p["alignn"]:
        h, y = _egc(h, y, src, dst, lp["node"], "crystal", BN, BE)
        y, z = _egc(y, z, lsrc, ldst, lp["edge"], "line", BE, BL)
    for lp in p["gcn"]:
        h, y = _egc(h, y, src, dst, lp, "crystal", BN, BE)

    out = _pool_fc(h, p["fc_W"], _row(p["fc_b"]), BN)
    return jnp.squeeze(out)
